# baseline (speedup 1.0000x reference)
import functools

import jax
import jax.numpy as jnp
from jax import lax
from jax.experimental import pallas as pl
from jax.experimental.pallas import tpu as pltpu
from jax.experimental.pallas import tpu_sc as plsc

NUM_CONTEXTS = 1000000
CTX_DIM = 64
BATCH = 16384
N_FIELDS = 26
HIST = 200

NC = 2
NS = 16
L = 16
NW = NC * NS

B1 = BATCH * N_FIELDS
PER_W1 = B1 // NW

CC = 512
N_CHUNKS1 = PER_W1 // CC
GB = 128
N_BURSTS = CC // GB

SQ = BATCH // 4
SB = SQ // GB
UNITS_PER_W = HIST * 4 // NW


def _tanh16(x):
    e = jnp.exp(jnp.abs(x) * -2.0)
    t = (1.0 - e) / (1.0 + e)
    tb = plsc.bitcast(t, jnp.int32)
    sb = plsc.bitcast(x, jnp.int32) & jnp.int32(-2147483648)
    return plsc.bitcast(tb | sb, jnp.float32)


def _worker_id():
    return lax.axis_index("s") * NC + lax.axis_index("c")


def _fire_bursts(table, idx_v, rows_v, b, sem):
    for j in range(N_BURSTS):
        pltpu.async_copy(table.at[idx_v.at[b, j]],
                         rows_v.at[b, pl.ds(j * GB, GB)], sem)


def _wait_bursts(table, idx_v, rows_v, b, sem):
    for j in range(N_BURSTS):
        pltpu.make_async_copy(table.at[idx_v.at[b, j]],
                              rows_v.at[b, pl.ds(j * GB, GB)], sem).wait()


def _out1_slice(out1, wid, g):
    return out1.at[pl.ds(wid * PER_W1 + g * CC, CC)]


def _sc_body(topics2d, sa4, table, stable_pad, out1, out2,
             idx_v, rows_v, sidx_v, obuf, tt2, tt, sem, sem_o, sem_s):
    wid = _worker_id()

    pltpu.sync_copy(stable_pad, tt2)
    tt[...] = _tanh16(tt2[...])

    row0 = wid * (PER_W1 // GB)

    pltpu.sync_copy(topics2d.at[pl.ds(row0, N_BURSTS)], idx_v.at[0])
    _fire_bursts(table, idx_v, rows_v, 0, sem)

    @pl.loop(0, N_CHUNKS1)
    def _ctx_chunk(g):
        b = lax.rem(g, 2)
        nb = 1 - b

        @pl.when(g >= 1)
        def _drain_out():
            pltpu.make_async_copy(rows_v.at[nb], _out1_slice(out1, wid, g - 1),
                                  sem_o).wait()

        @pl.when(g + 1 < N_CHUNKS1)
        def _prefetch():
            r0 = row0 + (g + 1) * N_BURSTS
            pltpu.sync_copy(topics2d.at[pl.ds(r0, N_BURSTS)], idx_v.at[nb])
            _fire_bursts(table, idx_v, rows_v, nb, sem)

        _wait_bursts(table, idx_v, rows_v, b, sem)

        @pl.loop(0, CC, unroll=8)
        def _row(i):
            for s in range(CTX_DIM // L):
                sl = pl.ds(s * L, L)
                rows_v[b, i, sl] = _tanh16(rows_v[b, i, sl])

        pltpu.async_copy(rows_v.at[b], _out1_slice(out1, wid, g), sem_o)

    pltpu.make_async_copy(rows_v.at[lax.rem(N_CHUNKS1 - 1, 2)],
                          _out1_slice(out1, wid, N_CHUNKS1 - 1), sem_o).wait()

    def _unit_coords(g):
        h = g >> 2
        return h >> 3, h & 7, g & 3

    def _stage_sidx(g, bb):
        tr, r, q = _unit_coords(g)
        pltpu.async_copy(sa4.at[tr, pl.ds(q * SB, SB), r], sidx_v.at[bb],
                         sem_s)

    def _wait_sidx(g, bb):
        tr, r, q = _unit_coords(g)
        pltpu.make_async_copy(sa4.at[tr, pl.ds(q * SB, SB), r],
                              sidx_v.at[bb], sem_s).wait()

    def _out2_slice(out2, g):
        h = g >> 2
        q = g & 3
        return out2.at[pl.ds(h * (BATCH // GB) + q * SB, SB)]

    u0 = wid * UNITS_PER_W
    _stage_sidx(u0, 0)

    @pl.loop(0, UNITS_PER_W)
    def _str_unit(u):
        g = u0 + u
        b = lax.rem(u, 2)
        nb = 1 - b

        @pl.when(u >= 1)
        def _drain_out2():
            pltpu.make_async_copy(obuf.at[nb], _out2_slice(out2, g - 1),
                                  sem_s).wait()

        @pl.when(u + 1 < UNITS_PER_W)
        def _prefetch2():
            _stage_sidx(g + 1, nb)

        _wait_sidx(g, b)

        @pl.loop(0, SB)
        def _grp(i):
            for l in range(GB // L):
                g0 = sidx_v[b, i, pl.ds(l * L, L)] * 2
                sl = pl.ds(l * L, L)
                obuf[b, i, 0, sl] = plsc.load_gather(tt, [g0])
                obuf[b, i, 1, sl] = plsc.load_gather(tt, [g0 + 1])

        pltpu.async_copy(obuf.at[b], _out2_slice(out2, g), sem_s)

    pltpu.make_async_copy(obuf.at[lax.rem(UNITS_PER_W - 1, 2)],
                          _out2_slice(out2, u0 + UNITS_PER_W - 1),
                          sem_s).wait()


NBLK_ALL = NUM_CONTEXTS // GB
NBLK_RR = NBLK_ALL // NW * NW
BLK_WORDS = GB * CTX_DIM
TAIL0 = NBLK_ALL * GB


def _tr_body(tableT, tail_rows, t2, stage0, stage1, tbuf0, tbuf1,
             sem_i, sem_o):
    wid = _worker_id()
    base = lax.iota(jnp.int32, L) * CTX_DIM

    def _fire_in(blk, sb, w):
        pltpu.async_copy(tableT.at[:, pl.ds(blk * GB, w)],
                         sb.at[:, pl.ds(0, w)], sem_i)

    def _wait_in(blk, sb, w):
        pltpu.make_async_copy(tableT.at[:, pl.ds(blk * GB, w)],
                              sb.at[:, pl.ds(0, w)], sem_i).wait()

    def _scatter(sb, tbf, w):
        @pl.loop(0, CTX_DIM)
        def _d(d):
            for l0 in range(w // L):
                v = sb[d, pl.ds(l0 * L, L)]
                plsc.store_scatter(tbf, [base + (l0 * L * CTX_DIM + d)], v)

    def _fire_out(blk, tbf, w):
        pltpu.async_copy(tbf.at[pl.ds(0, w * CTX_DIM)],
                         t2.at[pl.ds(blk * BLK_WORDS, w * CTX_DIM)], sem_o)

    def _wait_out(blk, tbf, w):
        pltpu.make_async_copy(tbf.at[pl.ds(0, w * CTX_DIM)],
                              t2.at[pl.ds(blk * BLK_WORDS, w * CTX_DIM)],
                              sem_o).wait()

    bufs = ((stage0, tbuf0), (stage1, tbuf1))
    n_rr = NBLK_RR // NW
    _fire_in(wid, stage0, GB)

    @pl.loop(0, n_rr // 2)
    def _blk2(v):
        for half in range(2):
            u = v * 2 + half
            sb, tbf = bufs[half]
            nsb = bufs[1 - half][0]
            blk = u * NW + wid
            _wait_in(blk, sb, GB)

            @pl.when(u + 1 < n_rr)
            def _pf():
                _fire_in((u + 1) * NW + wid, nsb, GB)

            @pl.when(u >= 2)
            def _dr():
                _wait_out((u - 2) * NW + wid, tbf, GB)

            _scatter(sb, tbf, GB)
            _fire_out(blk, tbf, GB)

    for u in (n_rr - 2, n_rr - 1):
        _wait_out(u * NW + wid, bufs[u % 2][1], GB)

    @pl.when(wid < 4)
    def _left():
        blk = NBLK_RR + wid
        _fire_in(blk, stage0, GB)
        _wait_in(blk, stage0, GB)
        _scatter(stage0, tbuf0, GB)
        _fire_out(blk, tbuf0, GB)
        _wait_out(blk, tbuf0, GB)

    @pl.when(wid == 4)
    def _tail():
        pltpu.sync_copy(tail_rows, t2.at[pl.ds(TAIL0 * CTX_DIM,
                                               (NUM_CONTEXTS - TAIL0)
                                               * CTX_DIM)])


@jax.jit
def _transpose_table(tableT, tail_rows):
    mesh = plsc.VectorSubcoreMesh(core_axis_name="c", subcore_axis_name="s",
                                  num_cores=NC, num_subcores=NS)
    f = pl.kernel(
        _tr_body,
        out_type=jax.ShapeDtypeStruct((NUM_CONTEXTS * CTX_DIM,), jnp.float32),
        mesh=mesh,
        compiler_params=pltpu.CompilerParams(needs_layout_passes=False,
                                             use_tc_tiling_on_sc=True),
        scratch_types=[
            pltpu.VMEM((CTX_DIM, GB), jnp.float32),
            pltpu.VMEM((CTX_DIM, GB), jnp.float32),
            pltpu.VMEM((BLK_WORDS,), jnp.float32),
            pltpu.VMEM((BLK_WORDS,), jnp.float32),
            pltpu.SemaphoreType.DMA,
            pltpu.SemaphoreType.DMA,
        ],
    )
    return f(tableT, tail_rows)


@jax.jit
def _run(topics2d, sa4, table, stable_pad):
    mesh = plsc.VectorSubcoreMesh(core_axis_name="c", subcore_axis_name="s",
                                  num_cores=NC, num_subcores=NS)
    f = pl.kernel(
        _sc_body,
        out_type=[
            jax.ShapeDtypeStruct((B1, CTX_DIM), jnp.float32),
            jax.ShapeDtypeStruct((HIST * BATCH // GB, 2, GB), jnp.float32),
        ],
        mesh=mesh,
        compiler_params=pltpu.CompilerParams(needs_layout_passes=False,
                                             use_tc_tiling_on_sc=False),
        scratch_types=[
            pltpu.VMEM((2, N_BURSTS, GB), jnp.int32),
            pltpu.VMEM((2, CC, CTX_DIM), jnp.float32),
            pltpu.VMEM((2, SB, GB), jnp.int32),
            pltpu.VMEM((2, SB, 2, GB), jnp.float32),
            pltpu.VMEM((L,), jnp.float32),
            pltpu.VMEM((L,), jnp.float32),
            pltpu.SemaphoreType.DMA,
            pltpu.SemaphoreType.DMA,
            pltpu.SemaphoreType.DMA,
        ],
    )
    return f(topics2d, sa4, table, stable_pad)


def kernel(topics, structure_abstracts, contextual_table, structural_table):
    topics2d = topics.reshape(B1 // GB, GB)
    sa4 = structure_abstracts.reshape(128, 128, 25, 8).transpose(2, 0, 3, 1)
    stable_pad = jnp.pad(structural_table.reshape(6), (0, L - 6))
    tail_rows = contextual_table[TAIL0:].reshape(-1)
    table = _transpose_table(contextual_table.T,
                             tail_rows).reshape(NUM_CONTEXTS, CTX_DIM)
    out1, out2 = _run(topics2d, sa4, table, stable_pad)
    out2 = out2.reshape(HIST, BATCH // GB, 2, GB).transpose(1, 3, 0, 2)
    return (out1.reshape(BATCH, 1, N_FIELDS * CTX_DIM),
            out2.reshape(BATCH, HIST, 2))


# speedup vs baseline: 1.2985x; 1.2985x over previous
import functools

import jax
import jax.numpy as jnp
from jax import lax
from jax.experimental import pallas as pl
from jax.experimental.pallas import tpu as pltpu
from jax.experimental.pallas import tpu_sc as plsc

NUM_CONTEXTS = 1000000
CTX_DIM = 64
BATCH = 16384
N_FIELDS = 26
HIST = 200

NC = 2
NS = 16
L = 16
NW = NC * NS

B1 = BATCH * N_FIELDS
PER_W1 = B1 // NW

CC = 512
N_CHUNKS1 = PER_W1 // CC
GB = 128
N_BURSTS = CC // GB

SQ = BATCH // 4
SB = SQ // GB
UNITS_PER_W = HIST * 4 // NW


def _tanh16(x):
    e = jnp.exp(jnp.abs(x) * -2.0)
    t = (1.0 - e) / (1.0 + e)
    tb = plsc.bitcast(t, jnp.int32)
    sb = plsc.bitcast(x, jnp.int32) & jnp.int32(-2147483648)
    return plsc.bitcast(tb | sb, jnp.float32)


def _worker_id():
    return lax.axis_index("s") * NC + lax.axis_index("c")


def _fire_bursts(table, idx_v, rows_v, b, sem):
    for j in range(N_BURSTS):
        pltpu.async_copy(table.at[idx_v.at[b, j]],
                         rows_v.at[b, pl.ds(j * GB, GB)], sem)


def _wait_bursts(table, idx_v, rows_v, b, sem):
    for j in range(N_BURSTS):
        pltpu.make_async_copy(table.at[idx_v.at[b, j]],
                              rows_v.at[b, pl.ds(j * GB, GB)], sem).wait()


def _out1_slice(out1, wid, g):
    return out1.at[pl.ds(wid * PER_W1 + g * CC, CC)]


def _sc_body(topics2d, sa4, table, stable_pad, out1, out2,
             idx_v, rows_v, sidx_v, obuf, tt2, tt, sem, sem_o, sem_s):
    wid = _worker_id()

    pltpu.sync_copy(stable_pad, tt2)
    tt[...] = _tanh16(tt2[...])

    row0 = wid * (PER_W1 // GB)

    pltpu.sync_copy(topics2d.at[pl.ds(row0, N_BURSTS)], idx_v.at[0])
    _fire_bursts(table, idx_v, rows_v, 0, sem)

    @pl.loop(0, N_CHUNKS1)
    def _ctx_chunk(g):
        b = lax.rem(g, 2)
        nb = 1 - b

        @pl.when(g >= 1)
        def _drain_out():
            pltpu.make_async_copy(rows_v.at[nb], _out1_slice(out1, wid, g - 1),
                                  sem_o).wait()

        @pl.when(g + 1 < N_CHUNKS1)
        def _prefetch():
            r0 = row0 + (g + 1) * N_BURSTS
            pltpu.sync_copy(topics2d.at[pl.ds(r0, N_BURSTS)], idx_v.at[nb])
            _fire_bursts(table, idx_v, rows_v, nb, sem)

        _wait_bursts(table, idx_v, rows_v, b, sem)

        @pl.loop(0, CC, unroll=8)
        def _row(i):
            for s in range(CTX_DIM // L):
                sl = pl.ds(s * L, L)
                rows_v[b, i, sl] = _tanh16(rows_v[b, i, sl])

        pltpu.async_copy(rows_v.at[b], _out1_slice(out1, wid, g), sem_o)

    pltpu.make_async_copy(rows_v.at[lax.rem(N_CHUNKS1 - 1, 2)],
                          _out1_slice(out1, wid, N_CHUNKS1 - 1), sem_o).wait()

    def _unit_coords(g):
        h = g >> 2
        return h >> 3, h & 7, g & 3

    def _stage_sidx(g, bb):
        tr, r, q = _unit_coords(g)
        pltpu.async_copy(sa4.at[tr, pl.ds(q * SB, SB), r], sidx_v.at[bb],
                         sem_s)

    def _wait_sidx(g, bb):
        tr, r, q = _unit_coords(g)
        pltpu.make_async_copy(sa4.at[tr, pl.ds(q * SB, SB), r],
                              sidx_v.at[bb], sem_s).wait()

    def _out2_slice(out2, g):
        h = g >> 2
        q = g & 3
        return out2.at[pl.ds(h * (BATCH // GB) + q * SB, SB)]

    u0 = wid * UNITS_PER_W
    _stage_sidx(u0, 0)

    @pl.loop(0, UNITS_PER_W)
    def _str_unit(u):
        g = u0 + u
        b = lax.rem(u, 2)
        nb = 1 - b

        @pl.when(u >= 1)
        def _drain_out2():
            pltpu.make_async_copy(obuf.at[nb], _out2_slice(out2, g - 1),
                                  sem_s).wait()

        @pl.when(u + 1 < UNITS_PER_W)
        def _prefetch2():
            _stage_sidx(g + 1, nb)

        _wait_sidx(g, b)

        @pl.loop(0, SB)
        def _grp(i):
            for l in range(GB // L):
                g0 = sidx_v[b, i, pl.ds(l * L, L)] * 2
                sl = pl.ds(l * L, L)
                obuf[b, i, 0, sl] = plsc.load_gather(tt, [g0])
                obuf[b, i, 1, sl] = plsc.load_gather(tt, [g0 + 1])

        pltpu.async_copy(obuf.at[b], _out2_slice(out2, g), sem_s)

    pltpu.make_async_copy(obuf.at[lax.rem(UNITS_PER_W - 1, 2)],
                          _out2_slice(out2, u0 + UNITS_PER_W - 1),
                          sem_s).wait()


NBLK_ALL = NUM_CONTEXTS // GB
NBLK_RR = NBLK_ALL // NW * NW
BLK_WORDS = GB * CTX_DIM
TAIL0 = NBLK_ALL * GB


def _tr_body(tableT, tail_rows, t2, stage0, stage1, tbuf0, tbuf1,
             ctb0, ctb1, sem_i, sem_o):
    wid = _worker_id()
    PITCH = CTX_DIM + 1

    def _fire_in(blk, sb, w):
        pltpu.async_copy(tableT.at[:, pl.ds(blk * GB, w)],
                         sb.at[:, pl.ds(0, w)], sem_i)

    def _wait_in(blk, sb, w):
        pltpu.make_async_copy(tableT.at[:, pl.ds(blk * GB, w)],
                              sb.at[:, pl.ds(0, w)], sem_i).wait()

    ii = lax.iota(jnp.int32, L)
    base = ii * PITCH

    def _scatter(sb, tbf, ctb, w):
        @pl.loop(0, CTX_DIM, unroll=4)
        def _d(d):
            for l0 in range(w // L):
                v = sb[d, pl.ds(l0 * L, L)]
                plsc.store_scatter(tbf, [base + (l0 * L * PITCH + d)], v)

        @pl.loop(0, w, unroll=4)
        def _l(l):
            for s in range(CTX_DIM // L):
                ctb[pl.ds(l * CTX_DIM + s * L, L)] = (
                    tbf[pl.ds(l * PITCH + s * L, L)])

    def _fire_out(blk, ctb, w):
        pltpu.async_copy(ctb.at[pl.ds(0, w * CTX_DIM)],
                         t2.at[pl.ds(blk * BLK_WORDS, w * CTX_DIM)], sem_o)

    def _wait_out(blk, ctb, w):
        pltpu.make_async_copy(ctb.at[pl.ds(0, w * CTX_DIM)],
                              t2.at[pl.ds(blk * BLK_WORDS, w * CTX_DIM)],
                              sem_o).wait()

    bufs = ((stage0, tbuf0, ctb0), (stage1, tbuf1, ctb1))
    n_rr = NBLK_RR // NW
    _fire_in(wid, stage0, GB)

    @pl.loop(0, n_rr // 2)
    def _blk2(v):
        for half in range(2):
            u = v * 2 + half
            sb, tbf, ctb = bufs[half]
            nsb = bufs[1 - half][0]
            blk = u * NW + wid
            _wait_in(blk, sb, GB)

            @pl.when(u + 1 < n_rr)
            def _pf():
                _fire_in((u + 1) * NW + wid, nsb, GB)

            @pl.when(u >= 2)
            def _dr():
                _wait_out((u - 2) * NW + wid, ctb, GB)

            _scatter(sb, tbf, ctb, GB)
            _fire_out(blk, ctb, GB)

    for u in (n_rr - 2, n_rr - 1):
        _wait_out(u * NW + wid, bufs[u % 2][2], GB)

    @pl.when(wid < 4)
    def _left():
        blk = NBLK_RR + wid
        _fire_in(blk, stage0, GB)
        _wait_in(blk, stage0, GB)
        _scatter(stage0, tbuf0, ctb0, GB)
        _fire_out(blk, ctb0, GB)
        _wait_out(blk, ctb0, GB)

    @pl.when(wid == 4)
    def _tail():
        pltpu.sync_copy(tail_rows, stage0.at[pl.ds(0, 32)])

        @pl.loop(0, 32)
        def _row(i):
            for s in range(GB // L):
                p = i * GB + s * L + ii
                v = stage0[i, pl.ds(s * L, L)]
                plsc.store_scatter(tbuf0, [p + (p >> 6)], v)

        @pl.loop(0, 64, unroll=4)
        def _l(l):
            for s in range(CTX_DIM // L):
                ctb0[pl.ds(l * CTX_DIM + s * L, L)] = (
                    tbuf0[pl.ds(l * PITCH + s * L, L)])

        pltpu.sync_copy(ctb0.at[pl.ds(0, 64 * CTX_DIM)],
                        t2.at[pl.ds(TAIL0 * CTX_DIM, 64 * CTX_DIM)])


@jax.jit
def _transpose_table(tableT, tail_rows):
    mesh = plsc.VectorSubcoreMesh(core_axis_name="c", subcore_axis_name="s",
                                  num_cores=NC, num_subcores=NS)
    f = pl.kernel(
        _tr_body,
        out_type=jax.ShapeDtypeStruct((NUM_CONTEXTS * CTX_DIM,), jnp.float32),
        mesh=mesh,
        compiler_params=pltpu.CompilerParams(needs_layout_passes=False,
                                             use_tc_tiling_on_sc=True),
        scratch_types=[
            pltpu.VMEM((CTX_DIM, GB), jnp.float32),
            pltpu.VMEM((CTX_DIM, GB), jnp.float32),
            pltpu.VMEM((GB * (CTX_DIM + 1),), jnp.float32),
            pltpu.VMEM((GB * (CTX_DIM + 1),), jnp.float32),
            pltpu.VMEM((BLK_WORDS,), jnp.float32),
            pltpu.VMEM((BLK_WORDS,), jnp.float32),
            pltpu.SemaphoreType.DMA,
            pltpu.SemaphoreType.DMA,
        ],
    )
    return f(tableT, tail_rows)


@jax.jit
def _run(topics2d, sa4, table, stable_pad):
    mesh = plsc.VectorSubcoreMesh(core_axis_name="c", subcore_axis_name="s",
                                  num_cores=NC, num_subcores=NS)
    f = pl.kernel(
        _sc_body,
        out_type=[
            jax.ShapeDtypeStruct((B1, CTX_DIM), jnp.float32),
            jax.ShapeDtypeStruct((HIST * BATCH // GB, 2, GB), jnp.float32),
        ],
        mesh=mesh,
        compiler_params=pltpu.CompilerParams(needs_layout_passes=False,
                                             use_tc_tiling_on_sc=False),
        scratch_types=[
            pltpu.VMEM((2, N_BURSTS, GB), jnp.int32),
            pltpu.VMEM((2, CC, CTX_DIM), jnp.float32),
            pltpu.VMEM((2, SB, GB), jnp.int32),
            pltpu.VMEM((2, SB, 2, GB), jnp.float32),
            pltpu.VMEM((L,), jnp.float32),
            pltpu.VMEM((L,), jnp.float32),
            pltpu.SemaphoreType.DMA,
            pltpu.SemaphoreType.DMA,
            pltpu.SemaphoreType.DMA,
        ],
    )
    return f(topics2d, sa4, table, stable_pad)


def kernel(topics, structure_abstracts, contextual_table, structural_table):
    topics2d = topics.reshape(B1 // GB, GB)
    sa4 = structure_abstracts.reshape(128, 128, 25, 8).transpose(2, 0, 3, 1)
    stable_pad = jnp.pad(structural_table.reshape(6), (0, L - 6))
    tail_rows = contextual_table[TAIL0:].reshape(32, 2 * CTX_DIM)
    table = _transpose_table(contextual_table.T,
                             tail_rows).reshape(NUM_CONTEXTS, CTX_DIM)
    out1, out2 = _run(topics2d, sa4, table, stable_pad)
    out2 = out2.reshape(HIST, BATCH // GB, 2, GB).transpose(1, 3, 0, 2)
    return (out1.reshape(BATCH, 1, N_FIELDS * CTX_DIM),
            out2.reshape(BATCH, HIST, 2))


# speedup vs baseline: 1.7021x; 1.3108x over previous
import functools

import jax
import jax.numpy as jnp
from jax import lax
from jax.experimental import pallas as pl
from jax.experimental.pallas import tpu as pltpu
from jax.experimental.pallas import tpu_sc as plsc

NUM_CONTEXTS = 1000000
CTX_DIM = 64
BATCH = 16384
N_FIELDS = 26
HIST = 200

NC = 2
NS = 16
L = 16
NW = NC * NS

B1 = BATCH * N_FIELDS
PER_W1 = B1 // NW

CC = 512
N_CHUNKS1 = PER_W1 // CC
GB = 128
N_BURSTS = CC // GB

SQ = BATCH // 4
SB = SQ // GB
UNITS_PER_W = HIST * 4 // NW


def _tanh16(x):
    e = jnp.exp(jnp.abs(x) * -2.0)
    t = (1.0 - e) / (1.0 + e)
    tb = plsc.bitcast(t, jnp.int32)
    sb = plsc.bitcast(x, jnp.int32) & jnp.int32(-2147483648)
    return plsc.bitcast(tb | sb, jnp.float32)


def _worker_id():
    return lax.axis_index("s") * NC + lax.axis_index("c")


def _fire_bursts(table, idx_v, rows_v, b, sem):
    for j in range(N_BURSTS):
        pltpu.async_copy(table.at[idx_v.at[b, j]],
                         rows_v.at[b, pl.ds(j * GB, GB)], sem)


def _wait_bursts(table, idx_v, rows_v, b, sem):
    for j in range(N_BURSTS):
        pltpu.make_async_copy(table.at[idx_v.at[b, j]],
                              rows_v.at[b, pl.ds(j * GB, GB)], sem).wait()


def _out1_slice(out1, wid, g):
    return out1.at[pl.ds(wid * PER_W1 + g * CC, CC)]


def _sc_body(topics2d, sa4, table, stable_pad, out1, out2,
             idx_v, rows_v, sidx_v, obuf, tt2, tt, sem, sem_o, sem_s):
    wid = _worker_id()

    pltpu.sync_copy(stable_pad, tt2)
    tt[...] = _tanh16(tt2[...])

    row0 = wid * (PER_W1 // GB)

    pltpu.sync_copy(topics2d.at[pl.ds(row0, N_BURSTS)], idx_v.at[0])
    _fire_bursts(table, idx_v, rows_v, 0, sem)

    @pl.loop(0, N_CHUNKS1)
    def _ctx_chunk(g):
        b = lax.rem(g, 2)
        nb = 1 - b

        @pl.when(g >= 1)
        def _drain_out():
            pltpu.make_async_copy(rows_v.at[nb], _out1_slice(out1, wid, g - 1),
                                  sem_o).wait()

        @pl.when(g + 1 < N_CHUNKS1)
        def _prefetch():
            r0 = row0 + (g + 1) * N_BURSTS
            pltpu.sync_copy(topics2d.at[pl.ds(r0, N_BURSTS)], idx_v.at[nb])
            _fire_bursts(table, idx_v, rows_v, nb, sem)

        _wait_bursts(table, idx_v, rows_v, b, sem)

        @pl.loop(0, CC, unroll=8)
        def _row(i):
            for s in range(CTX_DIM // L):
                sl = pl.ds(s * L, L)
                rows_v[b, i, sl] = _tanh16(rows_v[b, i, sl])

        pltpu.async_copy(rows_v.at[b], _out1_slice(out1, wid, g), sem_o)

    pltpu.make_async_copy(rows_v.at[lax.rem(N_CHUNKS1 - 1, 2)],
                          _out1_slice(out1, wid, N_CHUNKS1 - 1), sem_o).wait()

    def _unit_coords(g):
        h = g >> 2
        return h >> 3, h & 7, g & 3

    def _stage_sidx(g, bb):
        tr, r, q = _unit_coords(g)
        pltpu.async_copy(sa4.at[tr, pl.ds(q * SB, SB), r], sidx_v.at[bb],
                         sem_s)

    def _wait_sidx(g, bb):
        tr, r, q = _unit_coords(g)
        pltpu.make_async_copy(sa4.at[tr, pl.ds(q * SB, SB), r],
                              sidx_v.at[bb], sem_s).wait()

    def _out2_slice(out2, g):
        h = g >> 2
        q = g & 3
        return out2.at[pl.ds(h * (BATCH // GB) + q * SB, SB)]

    u0 = wid * UNITS_PER_W
    _stage_sidx(u0, 0)

    @pl.loop(0, UNITS_PER_W)
    def _str_unit(u):
        g = u0 + u
        b = lax.rem(u, 2)
        nb = 1 - b

        @pl.when(u >= 1)
        def _drain_out2():
            pltpu.make_async_copy(obuf.at[nb], _out2_slice(out2, g - 1),
                                  sem_s).wait()

        @pl.when(u + 1 < UNITS_PER_W)
        def _prefetch2():
            _stage_sidx(g + 1, nb)

        _wait_sidx(g, b)

        @pl.loop(0, SB)
        def _grp(i):
            for l in range(GB // L):
                g0 = sidx_v[b, i, pl.ds(l * L, L)] * 2
                sl = pl.ds(l * L, L)
                obuf[b, i, 0, sl] = plsc.load_gather(tt, [g0])
                obuf[b, i, 1, sl] = plsc.load_gather(tt, [g0 + 1])

        pltpu.async_copy(obuf.at[b], _out2_slice(out2, g), sem_s)

    pltpu.make_async_copy(obuf.at[lax.rem(UNITS_PER_W - 1, 2)],
                          _out2_slice(out2, u0 + UNITS_PER_W - 1),
                          sem_s).wait()


@jax.jit
def _run(topics2d, sa4, table, stable_pad):
    mesh = plsc.VectorSubcoreMesh(core_axis_name="c", subcore_axis_name="s",
                                  num_cores=NC, num_subcores=NS)
    f = pl.kernel(
        _sc_body,
        out_type=[
            jax.ShapeDtypeStruct((B1, CTX_DIM), jnp.float32),
            jax.ShapeDtypeStruct((HIST * BATCH // GB, 2, GB), jnp.float32),
        ],
        mesh=mesh,
        compiler_params=pltpu.CompilerParams(needs_layout_passes=False,
                                             use_tc_tiling_on_sc=False),
        scratch_types=[
            pltpu.VMEM((2, N_BURSTS, GB), jnp.int32),
            pltpu.VMEM((2, CC, CTX_DIM), jnp.float32),
            pltpu.VMEM((2, SB, GB), jnp.int32),
            pltpu.VMEM((2, SB, 2, GB), jnp.float32),
            pltpu.VMEM((L,), jnp.float32),
            pltpu.VMEM((L,), jnp.float32),
            pltpu.SemaphoreType.DMA,
            pltpu.SemaphoreType.DMA,
            pltpu.SemaphoreType.DMA,
        ],
    )
    return f(topics2d, sa4, table, stable_pad)


def kernel(topics, structure_abstracts, contextual_table, structural_table):
    topics2d = topics.reshape(B1 // GB, GB)
    sa4 = structure_abstracts.reshape(128, 128, 25, 8).transpose(2, 0, 3, 1)
    stable_pad = jnp.pad(structural_table.reshape(6), (0, L - 6))
    out1, out2 = _run(topics2d, sa4, contextual_table, stable_pad)
    out2 = out2.reshape(HIST, BATCH // GB, 2, GB).transpose(1, 3, 0, 2)
    return (out1.reshape(BATCH, 1, N_FIELDS * CTX_DIM),
            out2.reshape(BATCH, HIST, 2))
